# baseline (speedup 1.0000x reference)
import functools

import jax
import jax.numpy as jnp
from jax import lax
from jax.experimental import pallas as pl
from jax.experimental.pallas import tpu as pltpu
from jax.experimental.pallas import tpu_sc as plsc

B, C1, C2, O, K = 4, 16, 32, 32, 7
VI, V = 163842, 40962
D = 128

NW = 32
NVW = 1312
VP = NW * NVW
CH = 16
IDXC = CH * K
NCH = NVW // CH

VB = 512
GRID_MM = VP // VB
GRID_OUT = (V + VB - 1) // VB


def _worker_id():
    info = plsc.get_sparse_core_info()
    return lax.axis_index("s") * info.num_cores + lax.axis_index("c")


NBUF = 6


@functools.lru_cache(maxsize=None)
def _build_pool_sc():
    mesh = plsc.VectorSubcoreMesh(core_axis_name="c", subcore_axis_name="s")

    @functools.partial(
        pl.kernel, mesh=mesh,
        out_type=jax.ShapeDtypeStruct((VP, D), jnp.float32),
        scratch_types=[
            pltpu.VMEM((NVW * K,), jnp.int32),
            pltpu.VMEM((NBUF, IDXC, D), jnp.float32),
            pltpu.VMEM((2, CH, D), jnp.float32),
            pltpu.SemaphoreType.DMA,
            pltpu.SemaphoreType.DMA,
        ],
    )
    def _pool_sc(xt_hbm, idx_hbm, out_hbm, idx_v, gbuf, obuf, sem_g, sem_w):
        w = _worker_id()
        obase = w * NVW
        pltpu.sync_copy(idx_hbm.at[pl.ds(w * (NVW * K), NVW * K)], idx_v)

        def gather(ch):
            slot = lax.rem(ch, NBUF)
            return pltpu.make_async_copy(
                xt_hbm.at[idx_v.at[pl.ds(ch * IDXC, IDXC)]], gbuf.at[slot], sem_g)

        def write(ch, oslot):
            return pltpu.make_async_copy(
                obuf.at[oslot], out_hbm.at[pl.ds(obase + ch * CH, CH)], sem_w)

        for g in range(NBUF - 1):
            gather(g).start()

        def chunk(ch, carry):
            slot = lax.rem(ch, NBUF)
            oslot = lax.rem(ch, 2)
            gather(ch).wait()

            @pl.when(ch >= 2)
            def _():
                write(ch, oslot).wait()

            inv_k = jnp.float32(1.0 / K)
            for v in range(CH):
                r0 = v * K
                for j in range(D // 16):
                    sl = pl.ds(j * 16, 16)
                    acc = gbuf[slot, r0, sl]
                    for k in range(1, K):
                        acc = acc + gbuf[slot, r0 + k, sl]
                    obuf[oslot, v, sl] = acc * inv_k
            write(ch, oslot).start()

            @pl.when(ch + NBUF - 1 < NCH)
            def _():
                gather(ch + NBUF - 1).start()

            return carry

        lax.fori_loop(0, NCH, chunk, 0)
        write(0, 0).wait()
        write(0, 0).wait()

    return _pool_sc


@functools.lru_cache(maxsize=None)
def _build_gather_sc(dtype):
    mesh = plsc.VectorSubcoreMesh(core_axis_name="c", subcore_axis_name="s")

    @functools.partial(
        pl.kernel, mesh=mesh,
        out_type=jax.ShapeDtypeStruct((K * VP, D), dtype),
        scratch_types=[
            pltpu.VMEM((NVW * K,), jnp.int32),
            pltpu.VMEM((NBUF, IDXC, D), dtype),
            pltpu.SemaphoreType.DMA,
            pltpu.SemaphoreType.DMA,
        ],
    )
    def _gather_sc(tab_hbm, idx_hbm, out_hbm, idx_v, gbuf, sem_g, sem_w):
        base = _worker_id() * (NVW * K)
        pltpu.sync_copy(idx_hbm.at[pl.ds(base, NVW * K)], idx_v)

        def gather(ch):
            slot = lax.rem(ch, NBUF)
            return pltpu.make_async_copy(
                tab_hbm.at[idx_v.at[pl.ds(ch * IDXC, IDXC)]], gbuf.at[slot], sem_g)

        def write(ch):
            slot = lax.rem(ch, NBUF)
            return pltpu.make_async_copy(
                gbuf.at[slot], out_hbm.at[pl.ds(base + ch * IDXC, IDXC)], sem_w)

        for g in range(NBUF - 1):
            gather(g).start()

        def chunk(ch, carry):
            gather(ch).wait()
            write(ch).start()

            @pl.when((ch >= 1) & (ch + NBUF - 1 < NCH))
            def _():
                write(ch - 1).wait()

            @pl.when(ch + NBUF - 1 < NCH)
            def _():
                gather(ch + NBUF - 1).start()

            return carry

        lax.fori_loop(0, NCH, chunk, 0)
        for _ in range(NBUF):
            write(0).wait()

        return None

    return _gather_sc


def _conv1_body(a_ref, m_ref, y_ref, s_ref, q_ref):
    i = pl.program_id(0)
    y = lax.dot_general(a_ref[0].astype(jnp.float32), m_ref[0],
                        (((1,), (1,)), ((), ())),
                        preferred_element_type=jnp.float32)
    for k in range(1, K):
        y += lax.dot_general(a_ref[k].astype(jnp.float32), m_ref[k],
                             (((1,), (1,)), ((), ())),
                             preferred_element_type=jnp.float32)
    y_ref[...] = y
    rows = lax.broadcasted_iota(jnp.int32, (VB, B * O), 0) + i * VB
    ym = jnp.where(rows < V, y, 0.0)
    ps = jnp.broadcast_to(jnp.sum(ym, axis=0, keepdims=True), (8, B * O))
    pq = jnp.broadcast_to(jnp.sum(ym * ym, axis=0, keepdims=True), (8, B * O))

    @pl.when(i == 0)
    def _():
        s_ref[...] = ps
        q_ref[...] = pq

    @pl.when(i > 0)
    def _():
        s_ref[...] += ps
        q_ref[...] += pq


def _conv1_tc(a, m):
    return pl.pallas_call(
        _conv1_body,
        grid=(GRID_MM,),
        in_specs=[
            pl.BlockSpec((K, VB, D), lambda i: (0, i, 0)),
            pl.BlockSpec((K, B * O, D), lambda i: (0, 0, 0)),
        ],
        out_specs=[
            pl.BlockSpec((VB, B * O), lambda i: (i, 0)),
            pl.BlockSpec((8, B * O), lambda i: (0, 0)),
            pl.BlockSpec((8, B * O), lambda i: (0, 0)),
        ],
        out_shape=[
            jax.ShapeDtypeStruct((VP, B * O), jnp.float32),
            jax.ShapeDtypeStruct((8, B * O), jnp.float32),
            jax.ShapeDtypeStruct((8, B * O), jnp.float32),
        ],
    )(a, m)


def _conv2_body(a_ref, m_ref, cs_ref, cb_ref, y_ref, s_ref, q_ref):
    i = pl.program_id(0)
    cs = cs_ref[0:1, :]
    cb = cb_ref[0:1, :]
    y = None
    for k in range(K):
        h = a_ref[k].astype(jnp.float32) * cs + cb
        h = jnp.where(h >= 0, h, 0.2 * h)
        p = lax.dot_general(m_ref[k], h, (((1,), (1,)), ((), ())),
                            preferred_element_type=jnp.float32)
        y = p if y is None else y + p
    y_ref[...] = y
    cols = lax.broadcasted_iota(jnp.int32, (B * O, VB), 1) + i * VB
    ym = jnp.where(cols < V, y, 0.0)
    ps = jnp.broadcast_to(jnp.sum(ym, axis=1, keepdims=True), (B * O, 8))
    pq = jnp.broadcast_to(jnp.sum(ym * ym, axis=1, keepdims=True), (B * O, 8))

    @pl.when(i == 0)
    def _():
        s_ref[...] = ps
        q_ref[...] = pq

    @pl.when(i > 0)
    def _():
        s_ref[...] += ps
        q_ref[...] += pq


def _conv2_tc(a, m, cs, cb):
    return pl.pallas_call(
        _conv2_body,
        grid=(GRID_MM,),
        in_specs=[
            pl.BlockSpec((K, VB, D), lambda i: (0, i, 0)),
            pl.BlockSpec((K, B * O, D), lambda i: (0, 0, 0)),
            pl.BlockSpec((8, D), lambda i: (0, 0)),
            pl.BlockSpec((8, D), lambda i: (0, 0)),
        ],
        out_specs=[
            pl.BlockSpec((B * O, VB), lambda i: (0, i)),
            pl.BlockSpec((B * O, 8), lambda i: (0, 0)),
            pl.BlockSpec((B * O, 8), lambda i: (0, 0)),
        ],
        out_shape=[
            jax.ShapeDtypeStruct((B * O, VP), jnp.float32),
            jax.ShapeDtypeStruct((B * O, 8), jnp.float32),
            jax.ShapeDtypeStruct((B * O, 8), jnp.float32),
        ],
    )(a, m, cs, cb)


def _final_body(y_ref, rs_ref, rb_ref, o_ref):
    h = y_ref[...] * rs_ref[:, 0:1] + rb_ref[:, 0:1]
    o_ref[...] = jnp.where(h >= 0, h, 0.2 * h)


def _final_tc(y, rs, rb):
    return pl.pallas_call(
        _final_body,
        grid=(GRID_OUT,),
        in_specs=[
            pl.BlockSpec((B * O, VB), lambda i: (0, i)),
            pl.BlockSpec((B * O, 8), lambda i: (0, 0)),
            pl.BlockSpec((B * O, 8), lambda i: (0, 0)),
        ],
        out_specs=pl.BlockSpec((B * O, VB), lambda i: (0, i)),
        out_shape=jax.ShapeDtypeStruct((B * O, V), jnp.float32),
    )(y, rs, rb)


def _stats_to_affine(s, q, gamma, beta):
    n = jnp.float32(B * V)
    s = s.reshape(B, O).sum(axis=0)
    q = q.reshape(B, O).sum(axis=0)
    mean = s / n
    var = q / n - mean * mean
    scale = gamma * lax.rsqrt(var + 1e-5)
    shift = beta - mean * scale
    return scale, shift


def kernel(x, conv_neigh_indices, down_neigh_indices, down_indices,
           W1, b1, g1, be1, W2, b2, g2, be2):
    del down_indices, b1, b2
    f32 = jnp.float32

    xt = jnp.pad(x.reshape(B * C1, VI).T, ((0, 0), (0, D - B * C1)))

    idx_down = jnp.pad(down_neigh_indices.astype(jnp.int32).reshape(-1),
                       (0, (VP - V) * K))
    idx_conv = jnp.pad(conv_neigh_indices.astype(jnp.int32).T,
                       ((0, 0), (0, VP - V))).reshape(-1)

    eye = jnp.eye(B, dtype=f32)
    m1 = jnp.einsum('ock,bd->kbodc', W1.reshape(O, C1, K), eye)
    m1 = jnp.pad(m1.reshape(K, B * O, B * C1), ((0, 0), (0, 0), (0, D - B * C1)))
    m2 = jnp.einsum('ock,bd->kbodc', W2.reshape(O, C2, K), eye).reshape(K, B * O, D)

    xp = _build_pool_sc()(xt, idx_down)

    gth1 = _build_gather_sc(jnp.float32)(xp, idx_conv)
    y1, s1, q1 = _conv1_tc(gth1.reshape(K, VP, D), m1)
    sc1, sh1 = _stats_to_affine(s1[0], q1[0], g1, be1)
    cs = jnp.broadcast_to(jnp.tile(sc1, B)[None, :], (8, D))
    cb = jnp.broadcast_to(jnp.tile(sh1, B)[None, :], (8, D))

    gth2 = _build_gather_sc(jnp.float32)(y1, idx_conv)
    y2, s2, q2 = _conv2_tc(gth2.reshape(K, VP, D), m2, cs, cb)
    sc2, sh2 = _stats_to_affine(s2[:, 0], q2[:, 0], g2, be2)
    rs = jnp.broadcast_to(jnp.tile(sc2, B)[:, None], (B * O, 8))
    rb = jnp.broadcast_to(jnp.tile(sh2, B)[:, None], (B * O, 8))

    out = _final_tc(y2, rs, rb)
    return out.reshape(B, O, V), None


# speedup vs baseline: 1.0238x; 1.0238x over previous
import functools

import jax
import jax.numpy as jnp
from jax import lax
from jax.experimental import pallas as pl
from jax.experimental.pallas import tpu as pltpu
from jax.experimental.pallas import tpu_sc as plsc

B, C1, C2, O, K = 4, 16, 32, 32, 7
VI, V = 163842, 40962
D = 128

NW = 32
NVW = 1312
VP = NW * NVW
CH = 16
IDXC = CH * K
NCH = NVW // CH

VB = 512
VPH = VP // 2
GRID_MM = VPH // VB
GRID_OUT = (V + VB - 1) // VB


def _worker_id():
    info = plsc.get_sparse_core_info()
    return lax.axis_index("s") * info.num_cores + lax.axis_index("c")


NBUF = 6


@functools.lru_cache(maxsize=None)
def _build_pool_sc(W):
    mesh = plsc.VectorSubcoreMesh(core_axis_name="c", subcore_axis_name="s")

    @functools.partial(
        pl.kernel, mesh=mesh,
        out_type=jax.ShapeDtypeStruct((VP, W), jnp.float32),
        scratch_types=[
            pltpu.VMEM((NVW * K,), jnp.int32),
            pltpu.VMEM((NBUF, IDXC, W), jnp.float32),
            pltpu.VMEM((2, CH, W), jnp.float32),
            pltpu.SemaphoreType.DMA,
            pltpu.SemaphoreType.DMA,
        ],
    )
    def _pool_sc(xt_hbm, idx_hbm, out_hbm, idx_v, gbuf, obuf, sem_g, sem_w):
        w = _worker_id()
        obase = w * NVW
        pltpu.sync_copy(idx_hbm.at[pl.ds(w * (NVW * K), NVW * K)], idx_v)

        def gather(ch):
            slot = lax.rem(ch, NBUF)
            return pltpu.make_async_copy(
                xt_hbm.at[idx_v.at[pl.ds(ch * IDXC, IDXC)]], gbuf.at[slot], sem_g)

        def write(ch, oslot):
            return pltpu.make_async_copy(
                obuf.at[oslot], out_hbm.at[pl.ds(obase + ch * CH, CH)], sem_w)

        for g in range(NBUF - 1):
            gather(g).start()

        def chunk(ch, carry):
            slot = lax.rem(ch, NBUF)
            oslot = lax.rem(ch, 2)
            gather(ch).wait()

            @pl.when(ch >= 2)
            def _():
                write(ch, oslot).wait()

            inv_k = jnp.float32(1.0 / K)
            for v in range(CH):
                r0 = v * K
                for j in range(W // 16):
                    sl = pl.ds(j * 16, 16)
                    acc = gbuf[slot, r0, sl]
                    for k in range(1, K):
                        acc = acc + gbuf[slot, r0 + k, sl]
                    obuf[oslot, v, sl] = acc * inv_k
            write(ch, oslot).start()

            @pl.when(ch + NBUF - 1 < NCH)
            def _():
                gather(ch + NBUF - 1).start()

            return carry

        lax.fori_loop(0, NCH, chunk, 0)
        write(0, 0).wait()
        write(0, 0).wait()

    return _pool_sc


@functools.lru_cache(maxsize=None)
def _build_gather_sc(dtype, W, NV):
    nvw = NV // NW
    nch = nvw * K // IDXC
    assert nch * IDXC == nvw * K
    mesh = plsc.VectorSubcoreMesh(core_axis_name="c", subcore_axis_name="s")

    @functools.partial(
        pl.kernel, mesh=mesh,
        out_type=jax.ShapeDtypeStruct((K * NV, W), dtype),
        scratch_types=[
            pltpu.VMEM((nvw * K,), jnp.int32),
            pltpu.VMEM((NBUF, IDXC, W), dtype),
            pltpu.SemaphoreType.DMA,
            pltpu.SemaphoreType.DMA,
        ],
    )
    def _gather_sc(tab_hbm, idx_hbm, out_hbm, idx_v, gbuf, sem_g, sem_w):
        base = _worker_id() * (nvw * K)
        pltpu.sync_copy(idx_hbm.at[pl.ds(base, nvw * K)], idx_v)

        def gather(ch):
            slot = lax.rem(ch, NBUF)
            return pltpu.make_async_copy(
                tab_hbm.at[idx_v.at[pl.ds(ch * IDXC, IDXC)]], gbuf.at[slot], sem_g)

        def write(ch):
            slot = lax.rem(ch, NBUF)
            return pltpu.make_async_copy(
                gbuf.at[slot], out_hbm.at[pl.ds(base + ch * IDXC, IDXC)], sem_w)

        for g in range(NBUF - 1):
            gather(g).start()

        def chunk(ch, carry):
            gather(ch).wait()
            write(ch).start()

            @pl.when((ch >= 1) & (ch + NBUF - 1 < nch))
            def _():
                write(ch - 1).wait()

            @pl.when(ch + NBUF - 1 < nch)
            def _():
                gather(ch + NBUF - 1).start()

            return carry

        lax.fori_loop(0, nch, chunk, 0)
        for _ in range(NBUF):
            write(0).wait()

        return None

    return _gather_sc


def _conv1_tc(a, m, base):
    def body(a_ref, m_ref, y_ref, s_ref, q_ref):
        i = pl.program_id(0)
        y = lax.dot_general(a_ref[0].astype(jnp.float32), m_ref[0],
                            (((1,), (1,)), ((), ())),
                            preferred_element_type=jnp.float32)
        for k in range(1, K):
            y += lax.dot_general(a_ref[k].astype(jnp.float32), m_ref[k],
                                 (((1,), (1,)), ((), ())),
                                 preferred_element_type=jnp.float32)
        y_ref[...] = y
        rows = lax.broadcasted_iota(jnp.int32, (VB, B * O), 0) + (base + i * VB)
        ym = jnp.where(rows < V, y, 0.0)
        ps = jnp.broadcast_to(jnp.sum(ym, axis=0, keepdims=True), (8, B * O))
        pq = jnp.broadcast_to(jnp.sum(ym * ym, axis=0, keepdims=True), (8, B * O))

        @pl.when(i == 0)
        def _():
            s_ref[...] = ps
            q_ref[...] = pq

        @pl.when(i > 0)
        def _():
            s_ref[...] += ps
            q_ref[...] += pq

    return pl.pallas_call(
        body,
        grid=(GRID_MM,),
        in_specs=[
            pl.BlockSpec((K, VB, D), lambda i: (0, i, 0)),
            pl.BlockSpec((K, B * O, D), lambda i: (0, 0, 0)),
        ],
        out_specs=[
            pl.BlockSpec((VB, B * O), lambda i: (i, 0)),
            pl.BlockSpec((8, B * O), lambda i: (0, 0)),
            pl.BlockSpec((8, B * O), lambda i: (0, 0)),
        ],
        out_shape=[
            jax.ShapeDtypeStruct((VPH, B * O), jnp.float32),
            jax.ShapeDtypeStruct((8, B * O), jnp.float32),
            jax.ShapeDtypeStruct((8, B * O), jnp.float32),
        ],
    )(a, m)


def _conv2_tc(a, m, cs, cb, base):
    def body(a_ref, m_ref, cs_ref, cb_ref, y_ref, s_ref, q_ref):
        i = pl.program_id(0)
        cs = cs_ref[0:1, :]
        cb = cb_ref[0:1, :]
        y = None
        for k in range(K):
            h = a_ref[k].astype(jnp.float32) * cs + cb
            h = jnp.where(h >= 0, h, 0.2 * h)
            p = lax.dot_general(m_ref[k], h, (((1,), (1,)), ((), ())),
                                preferred_element_type=jnp.float32)
            y = p if y is None else y + p
        y_ref[...] = y
        cols = lax.broadcasted_iota(jnp.int32, (B * O, VB), 1) + (base + i * VB)
        ym = jnp.where(cols < V, y, 0.0)
        ps = jnp.broadcast_to(jnp.sum(ym, axis=1, keepdims=True), (B * O, 8))
        pq = jnp.broadcast_to(jnp.sum(ym * ym, axis=1, keepdims=True), (B * O, 8))

        @pl.when(i == 0)
        def _():
            s_ref[...] = ps
            q_ref[...] = pq

        @pl.when(i > 0)
        def _():
            s_ref[...] += ps
            q_ref[...] += pq

    return pl.pallas_call(
        body,
        grid=(GRID_MM,),
        in_specs=[
            pl.BlockSpec((K, VB, D), lambda i: (0, i, 0)),
            pl.BlockSpec((K, B * O, D), lambda i: (0, 0, 0)),
            pl.BlockSpec((8, D), lambda i: (0, 0)),
            pl.BlockSpec((8, D), lambda i: (0, 0)),
        ],
        out_specs=[
            pl.BlockSpec((B * O, VB), lambda i: (0, i)),
            pl.BlockSpec((B * O, 8), lambda i: (0, 0)),
            pl.BlockSpec((B * O, 8), lambda i: (0, 0)),
        ],
        out_shape=[
            jax.ShapeDtypeStruct((B * O, VPH), jnp.float32),
            jax.ShapeDtypeStruct((B * O, 8), jnp.float32),
            jax.ShapeDtypeStruct((B * O, 8), jnp.float32),
        ],
    )(a, m, cs, cb)


def _final_body(y_ref, rs_ref, rb_ref, o_ref):
    h = y_ref[...] * rs_ref[:, 0:1] + rb_ref[:, 0:1]
    o_ref[...] = jnp.where(h >= 0, h, 0.2 * h)


def _final_tc(y, rs, rb):
    return pl.pallas_call(
        _final_body,
        grid=(GRID_OUT,),
        in_specs=[
            pl.BlockSpec((B * O, VB), lambda i: (0, i)),
            pl.BlockSpec((B * O, 8), lambda i: (0, 0)),
            pl.BlockSpec((B * O, 8), lambda i: (0, 0)),
        ],
        out_specs=pl.BlockSpec((B * O, VB), lambda i: (0, i)),
        out_shape=jax.ShapeDtypeStruct((B * O, V), jnp.float32),
    )(y, rs, rb)


def _stats_to_affine(s, q, gamma, beta):
    n = jnp.float32(B * V)
    s = s.reshape(B, O).sum(axis=0)
    q = q.reshape(B, O).sum(axis=0)
    mean = s / n
    var = q / n - mean * mean
    scale = gamma * lax.rsqrt(var + 1e-5)
    shift = beta - mean * scale
    return scale, shift


def kernel(x, conv_neigh_indices, down_neigh_indices, down_indices,
           W1, b1, g1, be1, W2, b2, g2, be2):
    del down_indices, b1, b2
    f32 = jnp.float32

    xt = jnp.pad(x.reshape(B * C1, VI).T, ((0, 0), (0, D - B * C1)))

    idx_down = jnp.pad(down_neigh_indices.astype(jnp.int32).reshape(-1),
                       (0, (VP - V) * K))
    ic = jnp.pad(conv_neigh_indices.astype(jnp.int32).T,
                 ((0, 0), (0, VP - V)))
    ica = ic[:, :VPH].reshape(-1)
    icb = ic[:, VPH:].reshape(-1)

    eye = jnp.eye(B, dtype=f32)
    m1 = jnp.einsum('ock,bd->kbodc', W1.reshape(O, C1, K), eye)
    m1 = jnp.pad(m1.reshape(K, B * O, B * C1), ((0, 0), (0, 0), (0, D - B * C1)))
    m2 = jnp.einsum('ock,bd->kbodc', W2.reshape(O, C2, K), eye).reshape(K, B * O, D)

    xp = _build_pool_sc(D)(xt, idx_down)

    gather = _build_gather_sc(jnp.float32, D, VPH)
    g1a = gather(xp, ica).reshape(K, VPH, D)
    y1a, s1a, q1a = _conv1_tc(g1a, m1, 0)
    g1b = gather(xp, icb).reshape(K, VPH, D)
    y1b, s1b, q1b = _conv1_tc(g1b, m1, VPH)
    y1 = jnp.concatenate([y1a, y1b], axis=0)
    sc1, sh1 = _stats_to_affine(s1a[0] + s1b[0], q1a[0] + q1b[0], g1, be1)
    cs = jnp.broadcast_to(jnp.tile(sc1, B)[None, :], (8, D))
    cb = jnp.broadcast_to(jnp.tile(sh1, B)[None, :], (8, D))

    g2a = gather(y1, ica).reshape(K, VPH, D)
    y2a, s2a, q2a = _conv2_tc(g2a, m2, cs, cb, 0)
    g2b = gather(y1, icb).reshape(K, VPH, D)
    y2b, s2b, q2b = _conv2_tc(g2b, m2, cs, cb, VPH)
    y2 = jnp.concatenate([y2a, y2b], axis=1)
    sc2, sh2 = _stats_to_affine(s2a[:, 0] + s2b[:, 0],
                                q2a[:, 0] + q2b[:, 0], g2, be2)
    rs = jnp.broadcast_to(jnp.tile(sc2, B)[:, None], (B * O, 8))
    rb = jnp.broadcast_to(jnp.tile(sh2, B)[:, None], (B * O, 8))

    out = _final_tc(y2, rs, rb)
    return out.reshape(B, O, V), None
